# baseline (speedup 1.0000x reference)
import functools

import jax
import jax.numpy as jnp
from jax import lax
from jax.experimental import pallas as pl
from jax.experimental.pallas import tpu as pltpu
from jax.experimental.pallas import tpu_sc as plsc

FDIM = 9
NB = 8
WIDTH = FDIM * NB
PAD = 80


def _sc_scatter_add(table, src, dst, width, N):
    E = src.shape[0]
    C = 128
    NCH = E // C
    NW = 32
    Q, R = NCH // NW, NCH % NW
    NJMAX = Q + 1
    NBUF = 4
    G = (NJMAX + NBUF - 1) // NBUF
    RPS = (N // 16) // 8 * 8
    REM = N - 16 * RPS
    NV = width // 16
    ZR = 104
    assert RPS % ZR == 0 and REM <= ZR

    src2 = src.reshape(NCH, C)
    dst2 = dst.reshape(NCH, C)
    mesh = plsc.VectorSubcoreMesh(core_axis_name="c", subcore_axis_name="s")
    gather = table is not None

    def body(*refs):
        if gather:
            t_h = refs[0]
            (src_h, dst_h, p_h, acc, sidx2, didx2, rows3,
             zbuf) = refs[1:9]
            sg = list(refs[9:9 + NBUF])
            ss = list(refs[9 + NBUF:9 + 2 * NBUF])
        else:
            (src_h, dst_h, p_h, acc, sidx2, didx2, rows3, zbuf,
             ss0) = refs
        c = lax.axis_index("c")
        s = lax.axis_index("s")
        w = s * 2 + c
        nj = jnp.where(w < R, Q + 1, Q)
        r0 = Q * w + jnp.minimum(w, R)

        pltpu.sync_copy(src_h.at[pl.ds(r0, Q)], sidx2.at[pl.ds(0, Q)])
        pltpu.sync_copy(dst_h.at[pl.ds(r0, Q)], didx2.at[pl.ds(0, Q)])
        if R:
            @pl.when(w < R)
            def _():
                pltpu.sync_copy(src_h.at[pl.ds(r0 + Q, 1)],
                                sidx2.at[pl.ds(Q, 1)])
                pltpu.sync_copy(dst_h.at[pl.ds(r0 + Q, 1)],
                                didx2.at[pl.ds(Q, 1)])

        def zb(i, carry):
            zbuf[i // NV, pl.ds((i % NV) * 16, 16)] = jnp.zeros((16,), jnp.float32)
            return carry

        lax.fori_loop(0, ZR * NV, zb, 0)

        def zcp(i, carry):
            pltpu.sync_copy(zbuf, acc.at[pl.ds(s * RPS + i * ZR, ZR)])
            return carry

        lax.fori_loop(0, RPS // ZR, zcp, 0)
        if REM:
            @pl.when(s == 0)
            def _():
                pltpu.sync_copy(zbuf.at[pl.ds(0, REM)],
                                acc.at[pl.ds(16 * RPS, REM)])
        if not gather:
            def ob(i, carry):
                rows3[0, i // NV, pl.ds((i % NV) * 16, 16)] = jnp.ones(
                    (16,), jnp.float32)
                return carry

            lax.fori_loop(0, C * NV, ob, 0)
        plsc.subcore_barrier()

        if gather:
            def grp(g, carry):
                for b in range(NBUF):
                    k = g * NBUF + b

                    @pl.when(jnp.logical_and(g > 0, k < nj))
                    def _():
                        pltpu.make_async_copy(
                            rows3.at[b], acc.at[didx2.at[k - NBUF]],
                            ss[b]).wait()

                    @pl.when(k < nj)
                    def _():
                        pltpu.async_copy(t_h.at[sidx2.at[k]], rows3.at[b],
                                         sg[b])
                for b in range(NBUF):
                    k = g * NBUF + b

                    @pl.when(k < nj)
                    def _():
                        pltpu.make_async_copy(t_h.at[sidx2.at[k]],
                                              rows3.at[b], sg[b]).wait()
                        pltpu.async_copy(rows3.at[b], acc.at[didx2.at[k]],
                                         ss[b], add=True)
                return carry

            lax.fori_loop(0, G, grp, 0)
            for b in range(NBUF):
                kmb = (nj - 1 - b) // NBUF * NBUF + b

                @pl.when(nj > b)
                def _():
                    pltpu.make_async_copy(rows3.at[b], acc.at[didx2.at[kmb]],
                                          ss[b]).wait()
        else:
            def it(k, carry):
                @pl.when(k < nj)
                def _():
                    pltpu.async_copy(rows3.at[0], acc.at[didx2.at[k]], ss0,
                                     add=True)
                return carry

            lax.fori_loop(0, NJMAX, it, 0)

            def dr(k, carry):
                @pl.when(k < nj)
                def _():
                    pltpu.make_async_copy(rows3.at[0], acc.at[didx2.at[k]],
                                          ss0).wait()
                return carry

            lax.fori_loop(0, NJMAX, dr, 0)

        plsc.subcore_barrier()
        pltpu.sync_copy(acc.at[pl.ds(s * RPS, RPS)],
                        p_h.at[c, pl.ds(s * RPS, RPS)])
        if REM:
            @pl.when(s == 0)
            def _():
                pltpu.sync_copy(acc.at[pl.ds(16 * RPS, REM)],
                                p_h.at[c, pl.ds(16 * RPS, REM)])

    scratch = [
        pltpu.VMEM_SHARED((N, width), jnp.float32),
        pltpu.VMEM((NJMAX, C), jnp.int32),
        pltpu.VMEM((NJMAX, C), jnp.int32),
        pltpu.VMEM((NBUF, C, width), jnp.float32),
        pltpu.VMEM((ZR, width), jnp.float32),
    ]
    scratch += [pltpu.SemaphoreType.DMA] * (2 * NBUF if gather else 1)
    run = pl.kernel(
        body,
        out_type=jax.ShapeDtypeStruct((2, N, width), jnp.float32),
        mesh=mesh,
        compiler_params=pltpu.CompilerParams(use_tc_tiling_on_sc=False),
        scratch_types=scratch,
    )
    if gather:
        return run(table, src2, dst2)
    return run(src2, dst2)


def _mlp1(net, w_t):
    M, K = w_t.shape
    BM = 9216

    def body(n_ref, w_ref, o_ref):
        o_ref[...] = lax.dot_general(
            w_ref[...], n_ref[...], (((1,), (1,)), ((), ())),
            preferred_element_type=jnp.float32)

    return pl.pallas_call(
        body,
        grid=((M + BM - 1) // BM,),
        in_specs=[pl.BlockSpec((NB, K), lambda i: (0, 0)),
                  pl.BlockSpec((BM, K), lambda i: (i, 0))],
        out_specs=pl.BlockSpec((BM, NB), lambda i: (i, 0)),
        out_shape=jax.ShapeDtypeStruct((M, NB), jnp.float32),
    )(net, w_t)


_TN = 1000


def _stage_first(degp, x72, bw1):
    N = x72.shape[0]

    def body(d_ref, x_ref, bw_ref, t_ref, dv_ref):
        dp = d_ref[...]
        dv = lax.rsqrt(dp[0, :, :8] + dp[1, :, :8] + 1.0)
        h = jnp.dot(x_ref[...], bw_ref[...], preferred_element_type=jnp.float32)
        dvb = jnp.concatenate([dv] * (PAD // 8), axis=1)
        t_ref[...] = h * dvb
        dv_ref[...] = dv

    return pl.pallas_call(
        body,
        grid=(N // _TN,),
        in_specs=[pl.BlockSpec((2, _TN, 16), lambda i: (0, i, 0)),
                  pl.BlockSpec((_TN, WIDTH), lambda i: (i, 0)),
                  pl.BlockSpec((WIDTH, PAD), lambda i: (0, 0))],
        out_specs=[pl.BlockSpec((_TN, PAD), lambda i: (i, 0)),
                   pl.BlockSpec((_TN, 8), lambda i: (i, 0))],
        out_shape=[jax.ShapeDtypeStruct((N, PAD), jnp.float32),
                   jax.ShapeDtypeStruct((N, 8), jnp.float32)],
    )(degp, x72, bw1)


def _stage_mid(p, t, dv8, bw, m80, bias):
    N = t.shape[0]

    def body(p_ref, t_ref, dv_ref, bw_ref, m_ref, b_ref, o_ref):
        ps = p_ref[...]
        ssum = ps[0] + ps[1] + t_ref[...]
        dvb = jnp.concatenate([dv_ref[...]] * (PAD // 8), axis=1)
        agg = ssum * dvb + b_ref[...]
        mu = jnp.dot(agg, m_ref[...], preferred_element_type=jnp.float32)
        var = jnp.dot(agg * agg, m_ref[...],
                      preferred_element_type=jnp.float32) - mu * mu
        x = jnp.maximum((agg - mu) * lax.rsqrt(var + 1e-5), 0.0)
        o_ref[...] = jnp.dot(x, bw_ref[...],
                             preferred_element_type=jnp.float32) * dvb

    return pl.pallas_call(
        body,
        grid=(N // _TN,),
        in_specs=[pl.BlockSpec((2, _TN, PAD), lambda i: (0, i, 0)),
                  pl.BlockSpec((_TN, PAD), lambda i: (i, 0)),
                  pl.BlockSpec((_TN, 8), lambda i: (i, 0)),
                  pl.BlockSpec((PAD, PAD), lambda i: (0, 0)),
                  pl.BlockSpec((PAD, PAD), lambda i: (0, 0)),
                  pl.BlockSpec((1, PAD), lambda i: (0, 0))],
        out_specs=pl.BlockSpec((_TN, PAD), lambda i: (i, 0)),
        out_shape=jax.ShapeDtypeStruct((N, PAD), jnp.float32),
    )(p, t, dv8, bw, m80, bias)


def _stage_last(p, t, dv8, bias):
    N = t.shape[0]

    def body(p_ref, t_ref, dv_ref, b_ref, o_ref):
        ps = p_ref[...]
        ssum = ps[0] + ps[1] + t_ref[...]
        dvb = jnp.concatenate([dv_ref[...]] * (PAD // 8), axis=1)
        agg = ssum * dvb + b_ref[...]
        o_ref[...] = jnp.tanh(agg[:, :WIDTH])

    return pl.pallas_call(
        body,
        grid=(N // _TN,),
        in_specs=[pl.BlockSpec((2, _TN, PAD), lambda i: (0, i, 0)),
                  pl.BlockSpec((_TN, PAD), lambda i: (i, 0)),
                  pl.BlockSpec((_TN, 8), lambda i: (i, 0)),
                  pl.BlockSpec((1, PAD), lambda i: (0, 0))],
        out_specs=pl.BlockSpec((_TN, WIDTH), lambda i: (i, 0)),
        out_shape=jax.ShapeDtypeStruct((N, WIDTH), jnp.float32),
    )(p, t, dv8, bias)


def _kron8(w):
    return jnp.kron(w, jnp.eye(NB, dtype=w.dtype))


def _pad80(m):
    r, c = m.shape
    return jnp.pad(m, ((0, PAD - r), (0, PAD - c)))


def kernel(net, edge_index, W_mlp1, W1, b1, W2, b2, W3, b3):
    N = W_mlp1.shape[1] // FDIM
    src = edge_index[0]
    dst = edge_index[1]

    bw1 = jnp.pad(_kron8(W1), ((0, 0), (0, PAD - WIDTH)))
    bw2 = _pad80(_kron8(W2))
    bw3 = _pad80(_kron8(W3))
    m80 = _pad80(_kron8(jnp.full((FDIM, FDIM), 1.0 / FDIM, jnp.float32)))
    bias1 = jnp.pad(jnp.repeat(b1, NB), (0, PAD - WIDTH))[None, :]
    bias2 = jnp.pad(jnp.repeat(b2, NB), (0, PAD - WIDTH))[None, :]
    bias3 = jnp.pad(jnp.repeat(b3, NB), (0, PAD - WIDTH))[None, :]

    degp = _sc_scatter_add(None, dst, dst, 16, N)

    y = _mlp1(net, W_mlp1.T)
    x72 = y.reshape(N, WIDTH)

    t1, dv8 = _stage_first(degp, x72, bw1)
    p1 = _sc_scatter_add(t1, src, dst, PAD, N)
    t2 = _stage_mid(p1, t1, dv8, bw2, m80, bias1)
    p2 = _sc_scatter_add(t2, src, dst, PAD, N)
    t3 = _stage_mid(p2, t2, dv8, bw3, m80, bias2)
    p3 = _sc_scatter_add(t3, src, dst, PAD, N)
    t4 = _stage_mid(p3, t3, dv8, bw3, m80, bias3)
    p4 = _sc_scatter_add(t4, src, dst, PAD, N)
    out72 = _stage_last(p4, t4, dv8, bias3)

    return out72.reshape(N, FDIM, NB).transpose(2, 0, 1)


# speedup vs baseline: 1.0131x; 1.0131x over previous
import functools

import jax
import jax.numpy as jnp
from jax import lax
from jax.experimental import pallas as pl
from jax.experimental.pallas import tpu as pltpu
from jax.experimental.pallas import tpu_sc as plsc

FDIM = 9
NB = 8
WIDTH = FDIM * NB
PAD = 80


def _sc_scatter_add(table, src, dst, width, N):
    E = src.shape[0]
    C = 128
    NCH = E // C
    NW = 32
    Q, R = NCH // NW, NCH % NW
    NJMAX = Q + 1
    NBUF = 4
    G = (NJMAX + NBUF - 1) // NBUF
    RPS = (N // 16) // 8 * 8
    REM = N - 16 * RPS
    NV = width // 16
    ZR = 104
    assert RPS % ZR == 0 and REM <= ZR

    src2 = src.reshape(NCH, C)
    dst2 = dst.reshape(NCH, C)
    mesh = plsc.VectorSubcoreMesh(core_axis_name="c", subcore_axis_name="s")
    gather = table is not None

    def body(*refs):
        if gather:
            t_h = refs[0]
            (src_h, dst_h, p_h, acc, sidx2, didx2, rows3,
             zbuf) = refs[1:9]
            sg = list(refs[9:9 + NBUF])
            ss = list(refs[9 + NBUF:9 + 2 * NBUF])
        else:
            (src_h, dst_h, p_h, acc, sidx2, didx2, rows3, zbuf,
             ss0) = refs
        c = lax.axis_index("c")
        s = lax.axis_index("s")
        w = s * 2 + c
        nj = jnp.where(w < R, Q + 1, Q)
        r0 = Q * w + jnp.minimum(w, R)

        pltpu.sync_copy(src_h.at[pl.ds(r0, Q)], sidx2.at[pl.ds(0, Q)])
        pltpu.sync_copy(dst_h.at[pl.ds(r0, Q)], didx2.at[pl.ds(0, Q)])
        if R:
            @pl.when(w < R)
            def _():
                pltpu.sync_copy(src_h.at[pl.ds(r0 + Q, 1)],
                                sidx2.at[pl.ds(Q, 1)])
                pltpu.sync_copy(dst_h.at[pl.ds(r0 + Q, 1)],
                                didx2.at[pl.ds(Q, 1)])

        def zb(i, carry):
            zbuf[i // NV, pl.ds((i % NV) * 16, 16)] = jnp.zeros((16,), jnp.float32)
            return carry

        lax.fori_loop(0, ZR * NV, zb, 0)

        def zcp(i, carry):
            pltpu.sync_copy(zbuf, acc.at[pl.ds(s * RPS + i * ZR, ZR)])
            return carry

        if gather:
            @pl.when(c == 0)
            def _():
                pltpu.sync_copy(t_h.at[pl.ds(s * RPS, RPS)],
                                acc.at[pl.ds(s * RPS, RPS)])

            @pl.when(c != 0)
            def _():
                lax.fori_loop(0, RPS // ZR, zcp, 0)
        else:
            lax.fori_loop(0, RPS // ZR, zcp, 0)
        if REM:
            @pl.when(s == 0)
            def _():
                if gather:
                    @pl.when(c == 0)
                    def _():
                        pltpu.sync_copy(t_h.at[pl.ds(16 * RPS, REM)],
                                        acc.at[pl.ds(16 * RPS, REM)])

                    @pl.when(c != 0)
                    def _():
                        pltpu.sync_copy(zbuf.at[pl.ds(0, REM)],
                                        acc.at[pl.ds(16 * RPS, REM)])
                else:
                    pltpu.sync_copy(zbuf.at[pl.ds(0, REM)],
                                    acc.at[pl.ds(16 * RPS, REM)])
        if not gather:
            def ob(i, carry):
                rows3[0, i // NV, pl.ds((i % NV) * 16, 16)] = jnp.ones(
                    (16,), jnp.float32)
                return carry

            lax.fori_loop(0, C * NV, ob, 0)
        plsc.subcore_barrier()

        if gather:
            def grp(g, carry):
                for b in range(NBUF):
                    k = g * NBUF + b

                    @pl.when(jnp.logical_and(g > 0, k < nj))
                    def _():
                        pltpu.make_async_copy(
                            rows3.at[b], acc.at[didx2.at[k - NBUF]],
                            ss[b]).wait()

                    @pl.when(k < nj)
                    def _():
                        pltpu.async_copy(t_h.at[sidx2.at[k]], rows3.at[b],
                                         sg[b])
                for b in range(NBUF):
                    k = g * NBUF + b

                    @pl.when(k < nj)
                    def _():
                        pltpu.make_async_copy(t_h.at[sidx2.at[k]],
                                              rows3.at[b], sg[b]).wait()
                        pltpu.async_copy(rows3.at[b], acc.at[didx2.at[k]],
                                         ss[b], add=True)
                return carry

            lax.fori_loop(0, G, grp, 0)
            for b in range(NBUF):
                kmb = (nj - 1 - b) // NBUF * NBUF + b

                @pl.when(nj > b)
                def _():
                    pltpu.make_async_copy(rows3.at[b], acc.at[didx2.at[kmb]],
                                          ss[b]).wait()
        else:
            def it(k, carry):
                @pl.when(k < nj)
                def _():
                    pltpu.async_copy(rows3.at[0], acc.at[didx2.at[k]], ss0,
                                     add=True)
                return carry

            lax.fori_loop(0, NJMAX, it, 0)

            def dr(k, carry):
                @pl.when(k < nj)
                def _():
                    pltpu.make_async_copy(rows3.at[0], acc.at[didx2.at[k]],
                                          ss0).wait()
                return carry

            lax.fori_loop(0, NJMAX, dr, 0)

        plsc.subcore_barrier()
        pltpu.sync_copy(acc.at[pl.ds(s * RPS, RPS)],
                        p_h.at[c, pl.ds(s * RPS, RPS)])
        if REM:
            @pl.when(s == 0)
            def _():
                pltpu.sync_copy(acc.at[pl.ds(16 * RPS, REM)],
                                p_h.at[c, pl.ds(16 * RPS, REM)])

    scratch = [
        pltpu.VMEM_SHARED((N, width), jnp.float32),
        pltpu.VMEM((NJMAX, C), jnp.int32),
        pltpu.VMEM((NJMAX, C), jnp.int32),
        pltpu.VMEM((NBUF, C, width), jnp.float32),
        pltpu.VMEM((ZR, width), jnp.float32),
    ]
    scratch += [pltpu.SemaphoreType.DMA] * (2 * NBUF if gather else 1)
    run = pl.kernel(
        body,
        out_type=jax.ShapeDtypeStruct((2, N, width), jnp.float32),
        mesh=mesh,
        compiler_params=pltpu.CompilerParams(use_tc_tiling_on_sc=False),
        scratch_types=scratch,
    )
    if gather:
        return run(table, src2, dst2)
    return run(src2, dst2)


def _mlp1(net, w_t):
    M, K = w_t.shape
    BM = 9216

    def body(n_ref, w_ref, o_ref):
        o_ref[...] = lax.dot_general(
            w_ref[...], n_ref[...], (((1,), (1,)), ((), ())),
            preferred_element_type=jnp.float32)

    return pl.pallas_call(
        body,
        grid=((M + BM - 1) // BM,),
        in_specs=[pl.BlockSpec((NB, K), lambda i: (0, 0)),
                  pl.BlockSpec((BM, K), lambda i: (i, 0))],
        out_specs=pl.BlockSpec((BM, NB), lambda i: (i, 0)),
        out_shape=jax.ShapeDtypeStruct((M, NB), jnp.float32),
    )(net, w_t)


_TN = 2000


def _stage_first(degp, x72, bw1):
    N = x72.shape[0]

    def body(d_ref, x_ref, bw_ref, t_ref, dv_ref):
        dp = d_ref[...]
        dv = lax.rsqrt(dp[0, :, :8] + dp[1, :, :8] + 1.0)
        h = jnp.dot(x_ref[...], bw_ref[...], preferred_element_type=jnp.float32)
        dvb = jnp.concatenate([dv] * (PAD // 8), axis=1)
        t_ref[...] = h * dvb
        dv_ref[...] = dv

    return pl.pallas_call(
        body,
        grid=(N // _TN,),
        in_specs=[pl.BlockSpec((2, _TN, 16), lambda i: (0, i, 0)),
                  pl.BlockSpec((_TN, WIDTH), lambda i: (i, 0)),
                  pl.BlockSpec((WIDTH, PAD), lambda i: (0, 0))],
        out_specs=[pl.BlockSpec((_TN, PAD), lambda i: (i, 0)),
                   pl.BlockSpec((_TN, 8), lambda i: (i, 0))],
        out_shape=[jax.ShapeDtypeStruct((N, PAD), jnp.float32),
                   jax.ShapeDtypeStruct((N, 8), jnp.float32)],
    )(degp, x72, bw1)


def _stage_mid(p, dv8, bw, m80, bias):
    N = p.shape[1]

    def body(p_ref, dv_ref, bw_ref, m_ref, b_ref, o_ref):
        ps = p_ref[...]
        ssum = ps[0] + ps[1]
        dvb = jnp.concatenate([dv_ref[...]] * (PAD // 8), axis=1)
        agg = ssum * dvb + b_ref[...]
        mu = jnp.dot(agg, m_ref[...], preferred_element_type=jnp.float32)
        var = jnp.dot(agg * agg, m_ref[...],
                      preferred_element_type=jnp.float32) - mu * mu
        x = jnp.maximum((agg - mu) * lax.rsqrt(var + 1e-5), 0.0)
        o_ref[...] = jnp.dot(x, bw_ref[...],
                             preferred_element_type=jnp.float32) * dvb

    return pl.pallas_call(
        body,
        grid=(N // _TN,),
        in_specs=[pl.BlockSpec((2, _TN, PAD), lambda i: (0, i, 0)),
                  pl.BlockSpec((_TN, 8), lambda i: (i, 0)),
                  pl.BlockSpec((PAD, PAD), lambda i: (0, 0)),
                  pl.BlockSpec((PAD, PAD), lambda i: (0, 0)),
                  pl.BlockSpec((1, PAD), lambda i: (0, 0))],
        out_specs=pl.BlockSpec((_TN, PAD), lambda i: (i, 0)),
        out_shape=jax.ShapeDtypeStruct((N, PAD), jnp.float32),
    )(p, dv8, bw, m80, bias)


def _stage_last(p, dv8, bias):
    N = p.shape[1]

    def body(p_ref, dv_ref, b_ref, o_ref):
        ps = p_ref[...]
        ssum = ps[0] + ps[1]
        dvb = jnp.concatenate([dv_ref[...]] * (PAD // 8), axis=1)
        agg = ssum * dvb + b_ref[...]
        o_ref[...] = jnp.tanh(agg[:, :WIDTH])

    return pl.pallas_call(
        body,
        grid=(N // _TN,),
        in_specs=[pl.BlockSpec((2, _TN, PAD), lambda i: (0, i, 0)),
                  pl.BlockSpec((_TN, 8), lambda i: (i, 0)),
                  pl.BlockSpec((1, PAD), lambda i: (0, 0))],
        out_specs=pl.BlockSpec((_TN, WIDTH), lambda i: (i, 0)),
        out_shape=jax.ShapeDtypeStruct((N, WIDTH), jnp.float32),
    )(p, dv8, bias)


def _kron8(w):
    return jnp.kron(w, jnp.eye(NB, dtype=w.dtype))


def _pad80(m):
    r, c = m.shape
    return jnp.pad(m, ((0, PAD - r), (0, PAD - c)))


def kernel(net, edge_index, W_mlp1, W1, b1, W2, b2, W3, b3):
    N = W_mlp1.shape[1] // FDIM
    src = edge_index[0]
    dst = edge_index[1]

    bw1 = jnp.pad(_kron8(W1), ((0, 0), (0, PAD - WIDTH)))
    bw2 = _pad80(_kron8(W2))
    bw3 = _pad80(_kron8(W3))
    m80 = _pad80(_kron8(jnp.full((FDIM, FDIM), 1.0 / FDIM, jnp.float32)))
    bias1 = jnp.pad(jnp.repeat(b1, NB), (0, PAD - WIDTH))[None, :]
    bias2 = jnp.pad(jnp.repeat(b2, NB), (0, PAD - WIDTH))[None, :]
    bias3 = jnp.pad(jnp.repeat(b3, NB), (0, PAD - WIDTH))[None, :]

    degp = _sc_scatter_add(None, dst, dst, 16, N)

    y = _mlp1(net, W_mlp1.T)
    x72 = y.reshape(N, WIDTH)

    t1, dv8 = _stage_first(degp, x72, bw1)
    p1 = _sc_scatter_add(t1, src, dst, PAD, N)
    t2 = _stage_mid(p1, dv8, bw2, m80, bias1)
    p2 = _sc_scatter_add(t2, src, dst, PAD, N)
    t3 = _stage_mid(p2, dv8, bw3, m80, bias2)
    p3 = _sc_scatter_add(t3, src, dst, PAD, N)
    t4 = _stage_mid(p3, dv8, bw3, m80, bias3)
    p4 = _sc_scatter_add(t4, src, dst, PAD, N)
    out72 = _stage_last(p4, dv8, bias3)

    return out72.reshape(N, FDIM, NB).transpose(2, 0, 1)


# speedup vs baseline: 1.1015x; 1.0873x over previous
import functools

import jax
import jax.numpy as jnp
import numpy as np
from jax import lax
from jax.experimental import pallas as pl
from jax.experimental.pallas import tpu as pltpu
from jax.experimental.pallas import tpu_sc as plsc

FDIM = 9
NB = 8
WIDTH = FDIM * NB
PAD = 80

_SEL = np.tile(np.eye(NB, dtype=np.float32), (1, PAD // NB))


def _sc_scatter_add(table, src, dst, width, N):
    E = src.shape[0]
    C = 128
    NCH = E // C
    NW = 32
    Q, R = NCH // NW, NCH % NW
    NJMAX = Q + 1
    NBUF = 4
    G = (NJMAX + NBUF - 1) // NBUF
    RPS = (N // 16) // 8 * 8
    REM = N - 16 * RPS
    NV = width // 16
    ZR = 104
    assert RPS % ZR == 0 and REM <= ZR

    src2 = src.reshape(NCH, C)
    dst2 = dst.reshape(NCH, C)
    mesh = plsc.VectorSubcoreMesh(core_axis_name="c", subcore_axis_name="s")
    gather = table is not None

    def body(*refs):
        if gather:
            t_h = refs[0]
            (src_h, dst_h, p_h, acc, sidx2, didx2, rows3,
             zbuf) = refs[1:9]
            sg = list(refs[9:9 + NBUF])
            ss = list(refs[9 + NBUF:9 + 2 * NBUF])
        else:
            (src_h, dst_h, p_h, acc, sidx2, didx2, rows3, zbuf,
             ss0) = refs
        c = lax.axis_index("c")
        s = lax.axis_index("s")
        w = s * 2 + c
        nj = jnp.where(w < R, Q + 1, Q)
        r0 = Q * w + jnp.minimum(w, R)

        pltpu.sync_copy(src_h.at[pl.ds(r0, Q)], sidx2.at[pl.ds(0, Q)])
        pltpu.sync_copy(dst_h.at[pl.ds(r0, Q)], didx2.at[pl.ds(0, Q)])
        if R:
            @pl.when(w < R)
            def _():
                pltpu.sync_copy(src_h.at[pl.ds(r0 + Q, 1)],
                                sidx2.at[pl.ds(Q, 1)])
                pltpu.sync_copy(dst_h.at[pl.ds(r0 + Q, 1)],
                                didx2.at[pl.ds(Q, 1)])

        def zb(i, carry):
            zbuf[i // NV, pl.ds((i % NV) * 16, 16)] = jnp.zeros((16,), jnp.float32)
            return carry

        lax.fori_loop(0, ZR * NV, zb, 0)

        def zcp(i, carry):
            pltpu.sync_copy(zbuf, acc.at[pl.ds(s * RPS + i * ZR, ZR)])
            return carry

        if gather:
            @pl.when(c == 0)
            def _():
                pltpu.sync_copy(t_h.at[pl.ds(s * RPS, RPS)],
                                acc.at[pl.ds(s * RPS, RPS)])

            @pl.when(c != 0)
            def _():
                lax.fori_loop(0, RPS // ZR, zcp, 0)
        else:
            lax.fori_loop(0, RPS // ZR, zcp, 0)
        if REM:
            @pl.when(s == 0)
            def _():
                if gather:
                    @pl.when(c == 0)
                    def _():
                        pltpu.sync_copy(t_h.at[pl.ds(16 * RPS, REM)],
                                        acc.at[pl.ds(16 * RPS, REM)])

                    @pl.when(c != 0)
                    def _():
                        pltpu.sync_copy(zbuf.at[pl.ds(0, REM)],
                                        acc.at[pl.ds(16 * RPS, REM)])
                else:
                    pltpu.sync_copy(zbuf.at[pl.ds(0, REM)],
                                    acc.at[pl.ds(16 * RPS, REM)])
        if not gather:
            def ob(i, carry):
                rows3[0, i // NV, pl.ds((i % NV) * 16, 16)] = jnp.ones(
                    (16,), jnp.float32)
                return carry

            lax.fori_loop(0, C * NV, ob, 0)
        plsc.subcore_barrier()

        if gather:
            def grp(g, carry):
                for b in range(NBUF):
                    k = g * NBUF + b

                    @pl.when(jnp.logical_and(g > 0, k < nj))
                    def _():
                        pltpu.make_async_copy(
                            rows3.at[b], acc.at[didx2.at[k - NBUF]],
                            ss[b]).wait()

                    @pl.when(k < nj)
                    def _():
                        pltpu.async_copy(t_h.at[sidx2.at[k]], rows3.at[b],
                                         sg[b])
                for b in range(NBUF):
                    k = g * NBUF + b

                    @pl.when(k < nj)
                    def _():
                        pltpu.make_async_copy(t_h.at[sidx2.at[k]],
                                              rows3.at[b], sg[b]).wait()
                        pltpu.async_copy(rows3.at[b], acc.at[didx2.at[k]],
                                         ss[b], add=True)
                return carry

            lax.fori_loop(0, G, grp, 0)
            for b in range(NBUF):
                kmb = (nj - 1 - b) // NBUF * NBUF + b

                @pl.when(nj > b)
                def _():
                    pltpu.make_async_copy(rows3.at[b], acc.at[didx2.at[kmb]],
                                          ss[b]).wait()
        else:
            def it(k, carry):
                @pl.when(k < nj)
                def _():
                    pltpu.async_copy(rows3.at[0], acc.at[didx2.at[k]], ss0,
                                     add=True)
                return carry

            lax.fori_loop(0, NJMAX, it, 0)

            def dr(k, carry):
                @pl.when(k < nj)
                def _():
                    pltpu.make_async_copy(rows3.at[0], acc.at[didx2.at[k]],
                                          ss0).wait()
                return carry

            lax.fori_loop(0, NJMAX, dr, 0)

        plsc.subcore_barrier()
        pltpu.sync_copy(acc.at[pl.ds(s * RPS, RPS)],
                        p_h.at[c, pl.ds(s * RPS, RPS)])
        if REM:
            @pl.when(s == 0)
            def _():
                pltpu.sync_copy(acc.at[pl.ds(16 * RPS, REM)],
                                p_h.at[c, pl.ds(16 * RPS, REM)])

    scratch = [
        pltpu.VMEM_SHARED((N, width), jnp.float32),
        pltpu.VMEM((NJMAX, C), jnp.int32),
        pltpu.VMEM((NJMAX, C), jnp.int32),
        pltpu.VMEM((NBUF, C, width), jnp.float32),
        pltpu.VMEM((ZR, width), jnp.float32),
    ]
    scratch += [pltpu.SemaphoreType.DMA] * (2 * NBUF if gather else 1)
    run = pl.kernel(
        body,
        out_type=jax.ShapeDtypeStruct((2, N, width), jnp.float32),
        mesh=mesh,
        compiler_params=pltpu.CompilerParams(use_tc_tiling_on_sc=False),
        scratch_types=scratch,
    )
    if gather:
        return run(table, src2, dst2)
    return run(src2, dst2)


def _mlp1(net, w_t):
    M, K = w_t.shape
    BM = 9216

    def body(n_ref, w_ref, o_ref):
        o_ref[...] = lax.dot_general(
            w_ref[...], n_ref[...], (((1,), (1,)), ((), ())),
            preferred_element_type=jnp.float32)

    return pl.pallas_call(
        body,
        grid=((M + BM - 1) // BM,),
        in_specs=[pl.BlockSpec((NB, K), lambda i: (0, 0)),
                  pl.BlockSpec((BM, K), lambda i: (i, 0))],
        out_specs=pl.BlockSpec((BM, NB), lambda i: (i, 0)),
        out_shape=jax.ShapeDtypeStruct((M, NB), jnp.float32),
    )(net, w_t)


_TN = 2000


def _stage_first(degp, x72, bw1):
    N = x72.shape[0]

    def body(d_ref, x_ref, bw_ref, sel_ref, t_ref, dv_ref):
        dp = d_ref[...]
        dv = lax.rsqrt(dp[0, :, :8] + dp[1, :, :8] + 1.0)
        h = jnp.dot(x_ref[...], bw_ref[...], preferred_element_type=jnp.float32)
        dvb = jnp.dot(dv, sel_ref[...], preferred_element_type=jnp.float32)
        t_ref[...] = h * dvb
        dv_ref[...] = dv

    return pl.pallas_call(
        body,
        grid=(N // _TN,),
        in_specs=[pl.BlockSpec((2, _TN, 16), lambda i: (0, i, 0)),
                  pl.BlockSpec((_TN, WIDTH), lambda i: (i, 0)),
                  pl.BlockSpec((WIDTH, PAD), lambda i: (0, 0)),
                  pl.BlockSpec((8, PAD), lambda i: (0, 0))],
        out_specs=[pl.BlockSpec((_TN, PAD), lambda i: (i, 0)),
                   pl.BlockSpec((_TN, 8), lambda i: (i, 0))],
        out_shape=[jax.ShapeDtypeStruct((N, PAD), jnp.float32),
                   jax.ShapeDtypeStruct((N, 8), jnp.float32)],
    )(degp, x72, bw1, _SEL)


def _stage_mid(p, dv8, bw, m80, bias):
    N = p.shape[1]

    def body(p_ref, dv_ref, bw_ref, m_ref, b_ref, sel_ref, o_ref):
        ps = p_ref[...]
        ssum = ps[0] + ps[1]
        dvb = jnp.dot(dv_ref[...], sel_ref[...],
                      preferred_element_type=jnp.float32)
        agg = ssum * dvb + b_ref[...]
        mu = jnp.dot(agg, m_ref[...], preferred_element_type=jnp.float32)
        var = jnp.dot(agg * agg, m_ref[...],
                      preferred_element_type=jnp.float32) - mu * mu
        x = jnp.maximum((agg - mu) * lax.rsqrt(var + 1e-5), 0.0)
        o_ref[...] = jnp.dot(x, bw_ref[...],
                             preferred_element_type=jnp.float32) * dvb

    return pl.pallas_call(
        body,
        grid=(N // _TN,),
        in_specs=[pl.BlockSpec((2, _TN, PAD), lambda i: (0, i, 0)),
                  pl.BlockSpec((_TN, 8), lambda i: (i, 0)),
                  pl.BlockSpec((PAD, PAD), lambda i: (0, 0)),
                  pl.BlockSpec((PAD, PAD), lambda i: (0, 0)),
                  pl.BlockSpec((1, PAD), lambda i: (0, 0)),
                  pl.BlockSpec((8, PAD), lambda i: (0, 0))],
        out_specs=pl.BlockSpec((_TN, PAD), lambda i: (i, 0)),
        out_shape=jax.ShapeDtypeStruct((N, PAD), jnp.float32),
    )(p, dv8, bw, m80, bias, _SEL)


def _stage_last(p, dv8, bias):
    N = p.shape[1]

    def body(p_ref, dv_ref, b_ref, sel_ref, o_ref):
        ps = p_ref[...]
        ssum = ps[0] + ps[1]
        dvb = jnp.dot(dv_ref[...], sel_ref[...],
                      preferred_element_type=jnp.float32)
        agg = ssum * dvb + b_ref[...]
        o_ref[...] = jnp.tanh(agg[:, :WIDTH])

    return pl.pallas_call(
        body,
        grid=(N // _TN,),
        in_specs=[pl.BlockSpec((2, _TN, PAD), lambda i: (0, i, 0)),
                  pl.BlockSpec((_TN, 8), lambda i: (i, 0)),
                  pl.BlockSpec((1, PAD), lambda i: (0, 0)),
                  pl.BlockSpec((8, PAD), lambda i: (0, 0))],
        out_specs=pl.BlockSpec((_TN, WIDTH), lambda i: (i, 0)),
        out_shape=jax.ShapeDtypeStruct((N, WIDTH), jnp.float32),
    )(p, dv8, bias, _SEL)


def _kron8(w):
    return jnp.kron(w, jnp.eye(NB, dtype=w.dtype))


def _pad80(m):
    r, c = m.shape
    return jnp.pad(m, ((0, PAD - r), (0, PAD - c)))


def kernel(net, edge_index, W_mlp1, W1, b1, W2, b2, W3, b3):
    N = W_mlp1.shape[1] // FDIM
    src = edge_index[0]
    dst = edge_index[1]

    bw1 = jnp.pad(_kron8(W1), ((0, 0), (0, PAD - WIDTH)))
    bw2 = _pad80(_kron8(W2))
    bw3 = _pad80(_kron8(W3))
    m80 = _pad80(_kron8(jnp.full((FDIM, FDIM), 1.0 / FDIM, jnp.float32)))
    bias1 = jnp.pad(jnp.repeat(b1, NB), (0, PAD - WIDTH))[None, :]
    bias2 = jnp.pad(jnp.repeat(b2, NB), (0, PAD - WIDTH))[None, :]
    bias3 = jnp.pad(jnp.repeat(b3, NB), (0, PAD - WIDTH))[None, :]

    degp = _sc_scatter_add(None, dst, dst, 16, N)

    y = _mlp1(net, W_mlp1.T)
    x72 = y.reshape(N, WIDTH)

    t1, dv8 = _stage_first(degp, x72, bw1)
    p1 = _sc_scatter_add(t1, src, dst, PAD, N)
    t2 = _stage_mid(p1, dv8, bw2, m80, bias1)
    p2 = _sc_scatter_add(t2, src, dst, PAD, N)
    t3 = _stage_mid(p2, dv8, bw3, m80, bias2)
    p3 = _sc_scatter_add(t3, src, dst, PAD, N)
    t4 = _stage_mid(p3, dv8, bw3, m80, bias3)
    p4 = _sc_scatter_add(t4, src, dst, PAD, N)
    out72 = _stage_last(p4, dv8, bias3)

    return out72.reshape(N, FDIM, NB).transpose(2, 0, 1)


# speedup vs baseline: 1.1021x; 1.0005x over previous
import functools

import jax
import jax.numpy as jnp
import numpy as np
from jax import lax
from jax.experimental import pallas as pl
from jax.experimental.pallas import tpu as pltpu
from jax.experimental.pallas import tpu_sc as plsc

FDIM = 9
NB = 8
WIDTH = FDIM * NB
PAD = 80

_SEL = np.tile(np.eye(NB, dtype=np.float32), (1, PAD // NB))


def _sc_scatter_add(table, src, dst, width, N):
    E = src.shape[0]
    C = 64
    NCH = E // C
    NW = 32
    Q, R = NCH // NW, NCH % NW
    NJMAX = Q + 1
    NBUF = 6
    G = (NJMAX + NBUF - 1) // NBUF
    RPS = (N // 16) // 8 * 8
    REM = N - 16 * RPS
    NV = width // 16
    ZR = 104
    assert RPS % ZR == 0 and REM <= ZR

    src2 = src.reshape(NCH, C)
    dst2 = dst.reshape(NCH, C)
    mesh = plsc.VectorSubcoreMesh(core_axis_name="c", subcore_axis_name="s")
    gather = table is not None

    def body(*refs):
        if gather:
            t_h = refs[0]
            (src_h, dst_h, p_h, acc, sidx2, didx2, rows3,
             zbuf) = refs[1:9]
            sg = list(refs[9:9 + NBUF])
            ss = list(refs[9 + NBUF:9 + 2 * NBUF])
        else:
            (src_h, dst_h, p_h, acc, sidx2, didx2, rows3, zbuf,
             ss0) = refs
        c = lax.axis_index("c")
        s = lax.axis_index("s")
        w = s * 2 + c
        nj = jnp.where(w < R, Q + 1, Q)
        r0 = Q * w + jnp.minimum(w, R)

        pltpu.sync_copy(src_h.at[pl.ds(r0, Q)], sidx2.at[pl.ds(0, Q)])
        pltpu.sync_copy(dst_h.at[pl.ds(r0, Q)], didx2.at[pl.ds(0, Q)])
        if R:
            @pl.when(w < R)
            def _():
                pltpu.sync_copy(src_h.at[pl.ds(r0 + Q, 1)],
                                sidx2.at[pl.ds(Q, 1)])
                pltpu.sync_copy(dst_h.at[pl.ds(r0 + Q, 1)],
                                didx2.at[pl.ds(Q, 1)])

        def zb(i, carry):
            zbuf[i // NV, pl.ds((i % NV) * 16, 16)] = jnp.zeros((16,), jnp.float32)
            return carry

        lax.fori_loop(0, ZR * NV, zb, 0)

        def zcp(i, carry):
            pltpu.sync_copy(zbuf, acc.at[pl.ds(s * RPS + i * ZR, ZR)])
            return carry

        if gather:
            @pl.when(c == 0)
            def _():
                pltpu.sync_copy(t_h.at[pl.ds(s * RPS, RPS)],
                                acc.at[pl.ds(s * RPS, RPS)])

            @pl.when(c != 0)
            def _():
                lax.fori_loop(0, RPS // ZR, zcp, 0)
        else:
            lax.fori_loop(0, RPS // ZR, zcp, 0)
        if REM:
            @pl.when(s == 0)
            def _():
                if gather:
                    @pl.when(c == 0)
                    def _():
                        pltpu.sync_copy(t_h.at[pl.ds(16 * RPS, REM)],
                                        acc.at[pl.ds(16 * RPS, REM)])

                    @pl.when(c != 0)
                    def _():
                        pltpu.sync_copy(zbuf.at[pl.ds(0, REM)],
                                        acc.at[pl.ds(16 * RPS, REM)])
                else:
                    pltpu.sync_copy(zbuf.at[pl.ds(0, REM)],
                                    acc.at[pl.ds(16 * RPS, REM)])
        if not gather:
            def ob(i, carry):
                rows3[0, i // NV, pl.ds((i % NV) * 16, 16)] = jnp.ones(
                    (16,), jnp.float32)
                return carry

            lax.fori_loop(0, C * NV, ob, 0)
        plsc.subcore_barrier()

        if gather:
            def grp(g, carry):
                for b in range(NBUF):
                    k = g * NBUF + b

                    @pl.when(jnp.logical_and(g > 0, k < nj))
                    def _():
                        pltpu.make_async_copy(
                            rows3.at[b], acc.at[didx2.at[k - NBUF]],
                            ss[b]).wait()

                    @pl.when(k < nj)
                    def _():
                        pltpu.async_copy(t_h.at[sidx2.at[k]], rows3.at[b],
                                         sg[b])
                for b in range(NBUF):
                    k = g * NBUF + b

                    @pl.when(k < nj)
                    def _():
                        pltpu.make_async_copy(t_h.at[sidx2.at[k]],
                                              rows3.at[b], sg[b]).wait()
                        pltpu.async_copy(rows3.at[b], acc.at[didx2.at[k]],
                                         ss[b], add=True)
                return carry

            lax.fori_loop(0, G, grp, 0)
            for b in range(NBUF):
                kmb = (nj - 1 - b) // NBUF * NBUF + b

                @pl.when(nj > b)
                def _():
                    pltpu.make_async_copy(rows3.at[b], acc.at[didx2.at[kmb]],
                                          ss[b]).wait()
        else:
            def it(k, carry):
                @pl.when(k < nj)
                def _():
                    pltpu.async_copy(rows3.at[0], acc.at[didx2.at[k]], ss0,
                                     add=True)
                return carry

            lax.fori_loop(0, NJMAX, it, 0)

            def dr(k, carry):
                @pl.when(k < nj)
                def _():
                    pltpu.make_async_copy(rows3.at[0], acc.at[didx2.at[k]],
                                          ss0).wait()
                return carry

            lax.fori_loop(0, NJMAX, dr, 0)

        plsc.subcore_barrier()
        pltpu.sync_copy(acc.at[pl.ds(s * RPS, RPS)],
                        p_h.at[c, pl.ds(s * RPS, RPS)])
        if REM:
            @pl.when(s == 0)
            def _():
                pltpu.sync_copy(acc.at[pl.ds(16 * RPS, REM)],
                                p_h.at[c, pl.ds(16 * RPS, REM)])

    scratch = [
        pltpu.VMEM_SHARED((N, width), jnp.float32),
        pltpu.VMEM((NJMAX, C), jnp.int32),
        pltpu.VMEM((NJMAX, C), jnp.int32),
        pltpu.VMEM((NBUF, C, width), jnp.float32),
        pltpu.VMEM((ZR, width), jnp.float32),
    ]
    scratch += [pltpu.SemaphoreType.DMA] * (2 * NBUF if gather else 1)
    run = pl.kernel(
        body,
        out_type=jax.ShapeDtypeStruct((2, N, width), jnp.float32),
        mesh=mesh,
        compiler_params=pltpu.CompilerParams(use_tc_tiling_on_sc=False),
        scratch_types=scratch,
    )
    if gather:
        return run(table, src2, dst2)
    return run(src2, dst2)


def _mlp1(net, w_t):
    M, K = w_t.shape
    BM = 9216

    def body(n_ref, w_ref, o_ref):
        o_ref[...] = lax.dot_general(
            w_ref[...], n_ref[...], (((1,), (1,)), ((), ())),
            preferred_element_type=jnp.float32)

    return pl.pallas_call(
        body,
        grid=((M + BM - 1) // BM,),
        in_specs=[pl.BlockSpec((NB, K), lambda i: (0, 0)),
                  pl.BlockSpec((BM, K), lambda i: (i, 0))],
        out_specs=pl.BlockSpec((BM, NB), lambda i: (i, 0)),
        out_shape=jax.ShapeDtypeStruct((M, NB), jnp.float32),
    )(net, w_t)


_TN = 2000


def _stage_first(degp, x72, bw1):
    N = x72.shape[0]

    def body(d_ref, x_ref, bw_ref, sel_ref, t_ref, dv_ref):
        dp = d_ref[...]
        dv = lax.rsqrt(dp[0, :, :8] + dp[1, :, :8] + 1.0)
        h = jnp.dot(x_ref[...], bw_ref[...], preferred_element_type=jnp.float32)
        dvb = jnp.dot(dv, sel_ref[...], preferred_element_type=jnp.float32)
        t_ref[...] = h * dvb
        dv_ref[...] = dv

    return pl.pallas_call(
        body,
        grid=(N // _TN,),
        in_specs=[pl.BlockSpec((2, _TN, 16), lambda i: (0, i, 0)),
                  pl.BlockSpec((_TN, WIDTH), lambda i: (i, 0)),
                  pl.BlockSpec((WIDTH, PAD), lambda i: (0, 0)),
                  pl.BlockSpec((8, PAD), lambda i: (0, 0))],
        out_specs=[pl.BlockSpec((_TN, PAD), lambda i: (i, 0)),
                   pl.BlockSpec((_TN, 8), lambda i: (i, 0))],
        out_shape=[jax.ShapeDtypeStruct((N, PAD), jnp.float32),
                   jax.ShapeDtypeStruct((N, 8), jnp.float32)],
    )(degp, x72, bw1, _SEL)


def _stage_mid(p, dv8, bw, m80, bias):
    N = p.shape[1]

    def body(p_ref, dv_ref, bw_ref, m_ref, b_ref, sel_ref, o_ref):
        ps = p_ref[...]
        ssum = ps[0] + ps[1]
        dvb = jnp.dot(dv_ref[...], sel_ref[...],
                      preferred_element_type=jnp.float32)
        agg = ssum * dvb + b_ref[...]
        mu = jnp.dot(agg, m_ref[...], preferred_element_type=jnp.float32)
        var = jnp.dot(agg * agg, m_ref[...],
                      preferred_element_type=jnp.float32) - mu * mu
        x = jnp.maximum((agg - mu) * lax.rsqrt(var + 1e-5), 0.0)
        o_ref[...] = jnp.dot(x, bw_ref[...],
                             preferred_element_type=jnp.float32) * dvb

    return pl.pallas_call(
        body,
        grid=(N // _TN,),
        in_specs=[pl.BlockSpec((2, _TN, PAD), lambda i: (0, i, 0)),
                  pl.BlockSpec((_TN, 8), lambda i: (i, 0)),
                  pl.BlockSpec((PAD, PAD), lambda i: (0, 0)),
                  pl.BlockSpec((PAD, PAD), lambda i: (0, 0)),
                  pl.BlockSpec((1, PAD), lambda i: (0, 0)),
                  pl.BlockSpec((8, PAD), lambda i: (0, 0))],
        out_specs=pl.BlockSpec((_TN, PAD), lambda i: (i, 0)),
        out_shape=jax.ShapeDtypeStruct((N, PAD), jnp.float32),
    )(p, dv8, bw, m80, bias, _SEL)


def _stage_last(p, dv8, bias):
    N = p.shape[1]

    def body(p_ref, dv_ref, b_ref, sel_ref, o_ref):
        ps = p_ref[...]
        ssum = ps[0] + ps[1]
        dvb = jnp.dot(dv_ref[...], sel_ref[...],
                      preferred_element_type=jnp.float32)
        agg = ssum * dvb + b_ref[...]
        o_ref[...] = jnp.tanh(agg[:, :WIDTH])

    return pl.pallas_call(
        body,
        grid=(N // _TN,),
        in_specs=[pl.BlockSpec((2, _TN, PAD), lambda i: (0, i, 0)),
                  pl.BlockSpec((_TN, 8), lambda i: (i, 0)),
                  pl.BlockSpec((1, PAD), lambda i: (0, 0)),
                  pl.BlockSpec((8, PAD), lambda i: (0, 0))],
        out_specs=pl.BlockSpec((_TN, WIDTH), lambda i: (i, 0)),
        out_shape=jax.ShapeDtypeStruct((N, WIDTH), jnp.float32),
    )(p, dv8, bias, _SEL)


def _kron8(w):
    return jnp.kron(w, jnp.eye(NB, dtype=w.dtype))


def _pad80(m):
    r, c = m.shape
    return jnp.pad(m, ((0, PAD - r), (0, PAD - c)))


def kernel(net, edge_index, W_mlp1, W1, b1, W2, b2, W3, b3):
    N = W_mlp1.shape[1] // FDIM
    src = edge_index[0]
    dst = edge_index[1]

    bw1 = jnp.pad(_kron8(W1), ((0, 0), (0, PAD - WIDTH)))
    bw2 = _pad80(_kron8(W2))
    bw3 = _pad80(_kron8(W3))
    m80 = _pad80(_kron8(jnp.full((FDIM, FDIM), 1.0 / FDIM, jnp.float32)))
    bias1 = jnp.pad(jnp.repeat(b1, NB), (0, PAD - WIDTH))[None, :]
    bias2 = jnp.pad(jnp.repeat(b2, NB), (0, PAD - WIDTH))[None, :]
    bias3 = jnp.pad(jnp.repeat(b3, NB), (0, PAD - WIDTH))[None, :]

    degp = _sc_scatter_add(None, dst, dst, 16, N)

    y = _mlp1(net, W_mlp1.T)
    x72 = y.reshape(N, WIDTH)

    t1, dv8 = _stage_first(degp, x72, bw1)
    p1 = _sc_scatter_add(t1, src, dst, PAD, N)
    t2 = _stage_mid(p1, dv8, bw2, m80, bias1)
    p2 = _sc_scatter_add(t2, src, dst, PAD, N)
    t3 = _stage_mid(p2, dv8, bw3, m80, bias2)
    p3 = _sc_scatter_add(t3, src, dst, PAD, N)
    t4 = _stage_mid(p3, dv8, bw3, m80, bias3)
    p4 = _sc_scatter_add(t4, src, dst, PAD, N)
    out72 = _stage_last(p4, dv8, bias3)

    return out72.reshape(N, FDIM, NB).transpose(2, 0, 1)
